# Initial kernel scaffold; baseline (speedup 1.0000x reference)
import jax
import jax.numpy as jnp
from jax.experimental import pallas as pl


def kernel(feat_c0, feat_c1):
    raise NotImplementedError("write your pallas kernel here")


# speedup vs baseline: 44.4171x; 44.4171x over previous
import functools

import jax
import jax.numpy as jnp
import numpy as np
from jax import lax
from jax.experimental import pallas as pl
from jax.experimental.pallas import tpu as pltpu
from jax.experimental.pallas import tpu_sc as plsc

L = 4096
LENGTH = 64
C = 256
TOPK = 3
RATIO = 0.85

BR = 256
NB = L // BR

_SC_CORES = 2
_SC_SUBCORES = 16
_SC_WORKERS = _SC_CORES * _SC_SUBCORES
_SC_LANES = 16
_PER_W = L // _SC_WORKERS
_VREGS_PER_W = _PER_W // _SC_LANES


def _border_mask_np():
    m = np.ones((LENGTH, LENGTH), dtype=np.float32)
    m[:2, :] = 0
    m[:, :2] = 0
    m[-2:, :] = 0
    m[:, -2:] = 0
    return m.reshape(-1)


def _topk_body(f1_ref, f2_ref, n1_ref, n2_ref,
               d1_ref, p1_ref, d2t_ref, p2t_ref,
               cval_ref, cidx_ref):
    i = pl.program_id(0)
    INF = jnp.float32(jnp.inf)
    BIG = jnp.int32(2**30)

    g = lax.dot_general(f1_ref[...], f2_ref[...],
                        (((1,), (1,)), ((), ())),
                        preferred_element_type=jnp.float32)
    d = (n1_ref[...][:, None] + n2_ref[...][None, :]) - 2.0 * g

    col_ids = lax.broadcasted_iota(jnp.int32, (BR, L), 1)
    row_ids = lax.broadcasted_iota(jnp.int32, (BR, L), 0)

    dw = d
    vals, idxs = [], []
    for _ in range(TOPK):
        m = jnp.min(dw, axis=1)
        im = jnp.min(jnp.where(dw == m[:, None], col_ids, L), axis=1)
        vals.append(m)
        idxs.append(im)
        dw = jnp.where(col_ids == im[:, None], INF, dw)
    d1_ref[...] = jnp.stack(vals, axis=1)
    p1_ref[...] = jnp.stack(idxs, axis=1)

    dw = d
    bvals, bidxs = [], []
    for _ in range(TOPK):
        m = jnp.min(dw, axis=0)
        im = jnp.min(jnp.where(dw == m[None, :], row_ids, BR), axis=0)
        bvals.append(m)
        bidxs.append(im + i * BR)
        dw = jnp.where(row_ids == im[None, :], INF, dw)
    bv = jnp.stack(bvals, axis=0)
    bi = jnp.stack(bidxs, axis=0)

    @pl.when(i == 0)
    def _():
        cval_ref[...] = jnp.full((TOPK, L), INF, jnp.float32)
        cidx_ref[...] = jnp.full((TOPK, L), BIG, jnp.int32)

    cat_v = jnp.concatenate([cval_ref[...], bv], axis=0)
    cat_i = jnp.concatenate([cidx_ref[...], bi], axis=0)
    nv, ni = [], []
    for _ in range(TOPK):
        m = jnp.min(cat_v, axis=0)
        sel = jnp.min(jnp.where(cat_v == m[None, :], cat_i, BIG), axis=0)
        nv.append(m)
        ni.append(sel)
        cat_v = jnp.where((cat_v == m[None, :]) & (cat_i == sel[None, :]),
                          INF, cat_v)
    cval_ref[...] = jnp.stack(nv, axis=0)
    cidx_ref[...] = jnp.stack(ni, axis=0)

    @pl.when(i == NB - 1)
    def _():
        d2t_ref[...] = cval_ref[...]
        p2t_ref[...] = cidx_ref[...]


_topk_call = pl.pallas_call(
    _topk_body,
    grid=(NB,),
    in_specs=[
        pl.BlockSpec((BR, C), lambda i: (i, 0)),
        pl.BlockSpec((L, C), lambda i: (0, 0)),
        pl.BlockSpec((BR,), lambda i: (i,)),
        pl.BlockSpec((L,), lambda i: (0,)),
    ],
    out_specs=[
        pl.BlockSpec((BR, TOPK), lambda i: (i, 0)),
        pl.BlockSpec((BR, TOPK), lambda i: (i, 0)),
        pl.BlockSpec((TOPK, L), lambda i: (0, 0)),
        pl.BlockSpec((TOPK, L), lambda i: (0, 0)),
    ],
    out_shape=[
        jax.ShapeDtypeStruct((L, TOPK), jnp.float32),
        jax.ShapeDtypeStruct((L, TOPK), jnp.int32),
        jax.ShapeDtypeStruct((TOPK, L), jnp.float32),
        jax.ShapeDtypeStruct((TOPK, L), jnp.int32),
    ],
    scratch_shapes=[
        pltpu.VMEM((TOPK, L), jnp.float32),
        pltpu.VMEM((TOPK, L), jnp.int32),
    ],
    compiler_params=pltpu.CompilerParams(
        dimension_semantics=("arbitrary",),
    ),
)


def _mutual_sc_body(r1n_hbm, r1d_hbm, r2n_hbm, r2d_hbm, j1_hbm, j2_hbm, mask_hbm,
               out_hbm,
               r1n_v, r1d_v, mask_v, j1_v, m1_v, r2n_v, r2d_v, j2_v, out_v):
    wid = lax.axis_index("s") * _SC_CORES + lax.axis_index("c")
    base = wid * _PER_W

    pltpu.sync_copy(r1n_hbm, r1n_v)
    pltpu.sync_copy(r1d_hbm, r1d_v)
    pltpu.sync_copy(mask_hbm, mask_v)
    pltpu.sync_copy(j1_hbm, j1_v)
    pltpu.sync_copy(r2n_hbm.at[pl.ds(base, _PER_W)], r2n_v)
    pltpu.sync_copy(r2d_hbm.at[pl.ds(base, _PER_W)], r2d_v)
    pltpu.sync_copy(j2_hbm.at[pl.ds(base, _PER_W)], j2_v)

    ratio_cut = jnp.float32(RATIO)
    zero = jnp.float32(0.0)
    one = jnp.float32(1.0)

    def m1_body(k, carry):
        sl = pl.ds(k * _SC_LANES, _SC_LANES)
        ratio = r1n_v[sl] / r1d_v[sl]
        res = jnp.where(ratio > ratio_cut, zero, ratio) * mask_v[sl]
        m1_v[sl] = jnp.where(res != zero, one, zero)
        return carry

    lax.fori_loop(0, L // _SC_LANES, m1_body, 0)

    lane_iota = lax.iota(jnp.int32, _SC_LANES)
    for s in range(_VREGS_PER_W):
        sl = pl.ds(s * _SC_LANES, _SC_LANES)
        j2s = j2_v[sl]
        g_m1 = plsc.load_gather(m1_v, [j2s])
        g_j1 = plsc.load_gather(j1_v, [j2s])
        ratio2 = r2n_v[sl] / r2d_v[sl]
        res2 = jnp.where(ratio2 > ratio_cut, zero, ratio2) * mask_v[
            pl.ds(base + s * _SC_LANES, _SC_LANES)]
        m2 = res2 != zero
        mut = m2 & (g_m1 != zero) & (g_j1 == (lane_iota + (base + s * _SC_LANES)))
        out_v[sl] = jnp.where(mut, 1, 0).astype(jnp.int32)

    pltpu.sync_copy(out_v, out_hbm.at[pl.ds(base, _PER_W)])


@functools.cache
def _mutual_sc():
    return pl.kernel(
        _mutual_sc_body,
        out_type=jax.ShapeDtypeStruct((L,), jnp.int32),
        mesh=plsc.VectorSubcoreMesh(core_axis_name="c", subcore_axis_name="s"),
        compiler_params=pltpu.CompilerParams(needs_layout_passes=False),
        scratch_types=[
            pltpu.VMEM((L,), jnp.float32),
            pltpu.VMEM((L,), jnp.float32),
            pltpu.VMEM((L,), jnp.float32),
            pltpu.VMEM((L,), jnp.int32),
            pltpu.VMEM((L,), jnp.float32),
            pltpu.VMEM((_PER_W,), jnp.float32),
            pltpu.VMEM((_PER_W,), jnp.float32),
            pltpu.VMEM((_PER_W,), jnp.int32),
            pltpu.VMEM((_PER_W,), jnp.int32),
        ],
    )


def kernel(feat_c0, feat_c1):
    scale = jnp.asarray(feat_c0.shape[-1], dtype=jnp.float32) ** 0.5
    f1 = (feat_c0 / scale)[0]
    f2 = (feat_c1 / scale)[0]
    n1 = jnp.sum(f1 * f1, axis=-1)
    n2 = jnp.sum(f2 * f2, axis=-1)

    distance1, preds1, d2t, p2t = _topk_call(f1, f2, n1, n2)
    distance2 = d2t.T
    preds2 = p2t.T

    mask = jnp.asarray(_border_mask_np())
    mutual_i32 = _mutual_sc()(
        distance1[:, 0], distance1[:, 1],
        distance2[:, 0], distance2[:, 1],
        preds1[:, 0], preds2[:, 0], mask,
    )
    mutual = mutual_i32.astype(bool)
    return distance1, preds1, distance2, preds2, mutual


# speedup vs baseline: 51.1189x; 1.1509x over previous
import functools

import jax
import jax.numpy as jnp
import numpy as np
from jax import lax
from jax.experimental import pallas as pl
from jax.experimental.pallas import tpu as pltpu
from jax.experimental.pallas import tpu_sc as plsc

L = 4096
LENGTH = 64
C = 256
TOPK = 3
RATIO = 0.85

BR = 256
NB = L // BR
CH = L // 128
ST = BR // 8

_SC_CORES = 2
_SC_LANES = 16
_SC_WORKERS = 32
_PER_W = L // _SC_WORKERS
_VREGS_PER_W = _PER_W // _SC_LANES


def _border_mask_np():
    m = np.ones((LENGTH, LENGTH), dtype=np.float32)
    m[:2, :] = 0
    m[:, :2] = 0
    m[-2:, :] = 0
    m[:, -2:] = 0
    return m.reshape(-1)


def _insert(x, xi, v1, i1, v2, i2, v3, i3):
    c1 = x < v1
    c2 = x < v2
    c3 = x < v3
    v3n = jnp.where(c3, jnp.where(c2, v2, x), v3)
    i3n = jnp.where(c3, jnp.where(c2, i2, xi), i3)
    v2n = jnp.where(c2, jnp.where(c1, v1, x), v2)
    i2n = jnp.where(c2, jnp.where(c1, i1, xi), i2)
    v1n = jnp.where(c1, x, v1)
    i1n = jnp.where(c1, xi, i1)
    return v1n, i1n, v2n, i2n, v3n, i3n


def _extract3(vals, gidx, axis):
    INF = jnp.float32(jnp.inf)
    BIG = jnp.int32(2**30)
    out_v, out_i = [], []
    for k in range(TOPK):
        m = jnp.min(vals, axis=axis)
        me = jnp.expand_dims(m, axis)
        sel = jnp.min(jnp.where(vals == me, gidx, BIG), axis=axis)
        out_v.append(m)
        out_i.append(sel)
        if k < TOPK - 1:
            sele = jnp.expand_dims(sel, axis)
            vals = jnp.where((vals == me) & (gidx == sele), INF, vals)
    return out_v, out_i


def _topk_body(f1_ref, f2_ref, n1_ref, n2_ref,
               d1_ref, p1_ref, d2t_ref, p2t_ref, d1t_ref, j1t_ref,
               cval_ref, cidx_ref):
    i = pl.program_id(0)
    INF = jnp.float32(jnp.inf)

    g = lax.dot_general(f1_ref[...], f2_ref[...],
                        (((1,), (1,)), ((), ())),
                        preferred_element_type=jnp.float32)
    d = (n1_ref[...][:, None] + n2_ref[...][None, :]) - g * jnp.float32(2.0**-7)

    lane128 = lax.broadcasted_iota(jnp.int32, (BR, 128), 1)
    zero_i = jnp.zeros((BR, 128), jnp.int32)
    v1 = d[:, 0:128]
    i1 = zero_i
    v2 = jnp.full((BR, 128), INF)
    i2 = zero_i
    v3 = jnp.full((BR, 128), INF)
    i3 = zero_i
    for c in range(1, CH):
        x = d[:, c * 128:(c + 1) * 128]
        xi = jnp.int32(c)
        v1, i1, v2, i2, v3, i3 = _insert(x, xi, v1, i1, v2, i2, v3, i3)
    rv = jnp.concatenate([v1, v2, v3], axis=1)
    rix = jnp.concatenate([i1 * 128 + lane128, i2 * 128 + lane128,
                           i3 * 128 + lane128], axis=1)
    vals, idxs = _extract3(rv, rix, axis=1)
    d1_ref[...] = jnp.stack(vals, axis=1)
    p1_ref[...] = jnp.stack(idxs, axis=1)
    d1t_ref[0, :] = vals[0]
    d1t_ref[1, :] = vals[1]
    j1t_ref[0, :] = idxs[0]

    sub8 = lax.broadcasted_iota(jnp.int32, (8, L), 0)
    zero_c = jnp.zeros((8, L), jnp.int32)
    w1 = d[0:8, :]
    j1c = zero_c
    w2 = jnp.full((8, L), INF)
    j2c = zero_c
    w3 = jnp.full((8, L), INF)
    j3c = zero_c
    for s in range(1, ST):
        x = d[s * 8:(s + 1) * 8, :]
        xi = jnp.int32(s)
        w1, j1c, w2, j2c, w3, j3c = _insert(x, xi, w1, j1c, w2, j2c, w3, j3c)

    @pl.when(i == 0)
    def _():
        cval_ref[...] = jnp.full((TOPK, L), INF, jnp.float32)
        cidx_ref[...] = jnp.full((TOPK, L), jnp.int32(2**30), jnp.int32)

    base = i * BR
    cat_v = jnp.concatenate([cval_ref[...], w1, w2, w3], axis=0)
    cat_i = jnp.concatenate([cidx_ref[...],
                             j1c * 8 + sub8 + base,
                             j2c * 8 + sub8 + base,
                             j3c * 8 + sub8 + base], axis=0)
    nv, ni = _extract3(cat_v, cat_i, axis=0)
    cval_ref[...] = jnp.stack(nv, axis=0)
    cidx_ref[...] = jnp.stack(ni, axis=0)

    @pl.when(i == NB - 1)
    def _():
        d2t_ref[...] = cval_ref[...]
        p2t_ref[...] = cidx_ref[...]


_topk_call = pl.pallas_call(
    _topk_body,
    grid=(NB,),
    in_specs=[
        pl.BlockSpec((BR, C), lambda i: (i, 0)),
        pl.BlockSpec((L, C), lambda i: (0, 0)),
        pl.BlockSpec((BR,), lambda i: (i,)),
        pl.BlockSpec((L,), lambda i: (0,)),
    ],
    out_specs=[
        pl.BlockSpec((BR, TOPK), lambda i: (i, 0)),
        pl.BlockSpec((BR, TOPK), lambda i: (i, 0)),
        pl.BlockSpec((TOPK, L), lambda i: (0, 0)),
        pl.BlockSpec((TOPK, L), lambda i: (0, 0)),
        pl.BlockSpec((2, BR), lambda i: (0, i)),
        pl.BlockSpec((1, BR), lambda i: (0, i)),
    ],
    out_shape=[
        jax.ShapeDtypeStruct((L, TOPK), jnp.float32),
        jax.ShapeDtypeStruct((L, TOPK), jnp.int32),
        jax.ShapeDtypeStruct((TOPK, L), jnp.float32),
        jax.ShapeDtypeStruct((TOPK, L), jnp.int32),
        jax.ShapeDtypeStruct((2, L), jnp.float32),
        jax.ShapeDtypeStruct((1, L), jnp.int32),
    ],
    scratch_shapes=[
        pltpu.VMEM((TOPK, L), jnp.float32),
        pltpu.VMEM((TOPK, L), jnp.int32),
    ],
    compiler_params=pltpu.CompilerParams(
        dimension_semantics=("arbitrary",),
    ),
)


def _mutual_sc_body(d1t_hbm, j1t_hbm, d2t_hbm, p2t_hbm, mask_hbm,
                    out_hbm,
                    r1n_v, r1d_v, mask_v, j1_v, m1_v, r2n_v, r2d_v, j2_v,
                    out_v):
    wid = lax.axis_index("s") * _SC_CORES + lax.axis_index("c")
    base = wid * _PER_W

    pltpu.sync_copy(d1t_hbm.at[0], r1n_v)
    pltpu.sync_copy(d1t_hbm.at[1], r1d_v)
    pltpu.sync_copy(mask_hbm, mask_v)
    pltpu.sync_copy(j1t_hbm.at[0], j1_v)
    pltpu.sync_copy(d2t_hbm.at[0, pl.ds(base, _PER_W)], r2n_v)
    pltpu.sync_copy(d2t_hbm.at[1, pl.ds(base, _PER_W)], r2d_v)
    pltpu.sync_copy(p2t_hbm.at[0, pl.ds(base, _PER_W)], j2_v)

    ratio_cut = jnp.float32(RATIO)
    zero = jnp.float32(0.0)
    one = jnp.float32(1.0)

    def m1_body(k, carry):
        sl = pl.ds(k * _SC_LANES, _SC_LANES)
        ratio = r1n_v[sl] / r1d_v[sl]
        res = jnp.where(ratio > ratio_cut, zero, ratio) * mask_v[sl]
        m1_v[sl] = jnp.where(res != zero, one, zero)
        return carry

    lax.fori_loop(0, L // _SC_LANES, m1_body, 0)

    lane_iota = lax.iota(jnp.int32, _SC_LANES)
    for s in range(_VREGS_PER_W):
        sl = pl.ds(s * _SC_LANES, _SC_LANES)
        j2s = j2_v[sl]
        g_m1 = plsc.load_gather(m1_v, [j2s])
        g_j1 = plsc.load_gather(j1_v, [j2s])
        ratio2 = r2n_v[sl] / r2d_v[sl]
        res2 = jnp.where(ratio2 > ratio_cut, zero, ratio2) * mask_v[
            pl.ds(base + s * _SC_LANES, _SC_LANES)]
        m2 = res2 != zero
        mut = m2 & (g_m1 != zero) & (g_j1 == (lane_iota + (base + s * _SC_LANES)))
        out_v[sl] = jnp.where(mut, 1, 0).astype(jnp.int32)

    pltpu.sync_copy(out_v, out_hbm.at[pl.ds(base, _PER_W)])


@functools.cache
def _mutual_sc():
    return pl.kernel(
        _mutual_sc_body,
        out_type=jax.ShapeDtypeStruct((L,), jnp.int32),
        mesh=plsc.VectorSubcoreMesh(core_axis_name="c", subcore_axis_name="s"),
        compiler_params=pltpu.CompilerParams(needs_layout_passes=False),
        scratch_types=[
            pltpu.VMEM((L,), jnp.float32),
            pltpu.VMEM((L,), jnp.float32),
            pltpu.VMEM((L,), jnp.float32),
            pltpu.VMEM((L,), jnp.int32),
            pltpu.VMEM((L,), jnp.float32),
            pltpu.VMEM((_PER_W,), jnp.float32),
            pltpu.VMEM((_PER_W,), jnp.float32),
            pltpu.VMEM((_PER_W,), jnp.int32),
            pltpu.VMEM((_PER_W,), jnp.int32),
        ],
    )


def kernel(feat_c0, feat_c1):
    scale = jnp.asarray(feat_c0.shape[-1], dtype=jnp.float32) ** 0.5
    f1r = feat_c0[0]
    f2r = feat_c1[0]
    f1 = f1r / scale
    f2 = f2r / scale
    n1 = jnp.sum(f1 * f1, axis=-1)
    n2 = jnp.sum(f2 * f2, axis=-1)

    distance1, preds1, d2t, p2t, d1t, j1t = _topk_call(f1r, f2r, n1, n2)
    distance2 = d2t.T
    preds2 = p2t.T

    mask = jnp.asarray(_border_mask_np())
    mutual_i32 = _mutual_sc()(d1t, j1t, d2t, p2t, mask)
    mutual = mutual_i32.astype(bool)
    return distance1, preds1, distance2, preds2, mutual


# speedup vs baseline: 56.0781x; 1.0970x over previous
import functools

import jax
import jax.numpy as jnp
import numpy as np
from jax import lax
from jax.experimental import pallas as pl
from jax.experimental.pallas import tpu as pltpu
from jax.experimental.pallas import tpu_sc as plsc

L = 4096
LENGTH = 64
C = 256
TOPK = 3
RATIO = 0.85

BR = 512
NB = L // BR
CH = L // 128
ST = BR // 8

_SC_CORES = 2
_SC_LANES = 16
_SC_WORKERS = 32
_PER_W = L // _SC_WORKERS
_VREGS_PER_W = _PER_W // _SC_LANES


def _border_mask_np():
    m = np.ones((LENGTH, LENGTH), dtype=np.float32)
    m[:2, :] = 0
    m[:, :2] = 0
    m[-2:, :] = 0
    m[:, -2:] = 0
    return m.reshape(-1)


def _insert(x, xi, v1, i1, v2, i2, v3, i3):
    c1 = x < v1
    c2 = x < v2
    c3 = x < v3
    v3n = jnp.where(c3, jnp.where(c2, v2, x), v3)
    i3n = jnp.where(c3, jnp.where(c2, i2, xi), i3)
    v2n = jnp.where(c2, jnp.where(c1, v1, x), v2)
    i2n = jnp.where(c2, jnp.where(c1, i1, xi), i2)
    v1n = jnp.where(c1, x, v1)
    i1n = jnp.where(c1, xi, i1)
    return v1n, i1n, v2n, i2n, v3n, i3n


def _extract3(vals, gidx, axis):
    INF = jnp.float32(jnp.inf)
    BIG = jnp.int32(2**30)
    out_v, out_i = [], []
    for k in range(TOPK):
        m = jnp.min(vals, axis=axis)
        me = jnp.expand_dims(m, axis)
        sel = jnp.min(jnp.where(vals == me, gidx, BIG), axis=axis)
        out_v.append(m)
        out_i.append(sel)
        if k < TOPK - 1:
            sele = jnp.expand_dims(sel, axis)
            vals = jnp.where(gidx == sele, INF, vals)
    return out_v, out_i


def _topk_body(f1_ref, f2_ref, n1_ref, n2_ref,
               d1_ref, p1_ref, d2t_ref, p2t_ref, d1t_ref, j1t_ref,
               cval_ref, cidx_ref):
    i = pl.program_id(0)
    INF = jnp.float32(jnp.inf)
    base = i * BR

    f1 = f1_ref[...]
    n1col = n1_ref[...][:, None]
    n2 = n2_ref[...]
    lane128 = lax.broadcasted_iota(jnp.int32, (BR, 128), 1)
    sub8 = lax.broadcasted_iota(jnp.int32, (8, 128), 0) + base

    v1 = i1 = v2 = i2 = v3 = i3 = None
    col_v, col_i = [], []
    for c in range(CH):
        f2c = f2_ref[c * 128:(c + 1) * 128, :]
        g = lax.dot_general(f1, f2c, (((1,), (1,)), ((), ())),
                            preferred_element_type=jnp.float32)
        dc = (n1col + n2[None, c * 128:(c + 1) * 128]) - g * jnp.float32(2.0**-7)

        if c == 0:
            zero_i = jnp.zeros((BR, 128), jnp.int32)
            v1, i1 = dc, zero_i
            v2, i2 = jnp.full((BR, 128), INF), zero_i
            v3, i3 = jnp.full((BR, 128), INF), zero_i
        else:
            v1, i1, v2, i2, v3, i3 = _insert(dc, jnp.int32(c),
                                             v1, i1, v2, i2, v3, i3)

        zero_c = jnp.zeros((8, 128), jnp.int32)
        w1, k1 = dc[0:8, :], zero_c
        w2, k2 = jnp.full((8, 128), INF), zero_c
        w3, k3 = jnp.full((8, 128), INF), zero_c
        for s in range(1, ST):
            w1, k1, w2, k2, w3, k3 = _insert(dc[s * 8:(s + 1) * 8, :],
                                             jnp.int32(s),
                                             w1, k1, w2, k2, w3, k3)
        col_v.append((w1, w2, w3))
        col_i.append((k1 * 8 + sub8, k2 * 8 + sub8, k3 * 8 + sub8))

    rv = jnp.concatenate([v1, v2, v3], axis=1)
    rix = jnp.concatenate([i1 * 128 + lane128, i2 * 128 + lane128,
                           i3 * 128 + lane128], axis=1)
    vals, idxs = _extract3(rv, rix, axis=1)
    d1_ref[...] = jnp.stack(vals, axis=1)
    p1_ref[...] = jnp.stack(idxs, axis=1)
    d1t_ref[0, :] = vals[0]
    d1t_ref[1, :] = vals[1]
    j1t_ref[0, :] = idxs[0]

    @pl.when(i == 0)
    def _():
        cval_ref[...] = jnp.full((TOPK, L), INF, jnp.float32)
        cidx_ref[...] = jnp.full((TOPK, L), jnp.int32(2**30), jnp.int32)

    wa = jnp.concatenate([w[0] for w in col_v], axis=1)
    wb = jnp.concatenate([w[1] for w in col_v], axis=1)
    wc = jnp.concatenate([w[2] for w in col_v], axis=1)
    ka = jnp.concatenate([k[0] for k in col_i], axis=1)
    kb = jnp.concatenate([k[1] for k in col_i], axis=1)
    kc = jnp.concatenate([k[2] for k in col_i], axis=1)
    cat_v = jnp.concatenate([cval_ref[...], wa, wb, wc], axis=0)
    cat_i = jnp.concatenate([cidx_ref[...], ka, kb, kc], axis=0)
    nv, ni = _extract3(cat_v, cat_i, axis=0)
    cval_ref[...] = jnp.stack(nv, axis=0)
    cidx_ref[...] = jnp.stack(ni, axis=0)

    @pl.when(i == NB - 1)
    def _():
        d2t_ref[...] = cval_ref[...]
        p2t_ref[...] = cidx_ref[...]


_topk_call = pl.pallas_call(
    _topk_body,
    grid=(NB,),
    in_specs=[
        pl.BlockSpec((BR, C), lambda i: (i, 0)),
        pl.BlockSpec((L, C), lambda i: (0, 0)),
        pl.BlockSpec((BR,), lambda i: (i,)),
        pl.BlockSpec((L,), lambda i: (0,)),
    ],
    out_specs=[
        pl.BlockSpec((BR, TOPK), lambda i: (i, 0)),
        pl.BlockSpec((BR, TOPK), lambda i: (i, 0)),
        pl.BlockSpec((TOPK, L), lambda i: (0, 0)),
        pl.BlockSpec((TOPK, L), lambda i: (0, 0)),
        pl.BlockSpec((2, BR), lambda i: (0, i)),
        pl.BlockSpec((1, BR), lambda i: (0, i)),
    ],
    out_shape=[
        jax.ShapeDtypeStruct((L, TOPK), jnp.float32),
        jax.ShapeDtypeStruct((L, TOPK), jnp.int32),
        jax.ShapeDtypeStruct((TOPK, L), jnp.float32),
        jax.ShapeDtypeStruct((TOPK, L), jnp.int32),
        jax.ShapeDtypeStruct((2, L), jnp.float32),
        jax.ShapeDtypeStruct((1, L), jnp.int32),
    ],
    scratch_shapes=[
        pltpu.VMEM((TOPK, L), jnp.float32),
        pltpu.VMEM((TOPK, L), jnp.int32),
    ],
    compiler_params=pltpu.CompilerParams(
        dimension_semantics=("arbitrary",),
    ),
)


def _mutual_sc_body(d1t_hbm, j1t_hbm, d2t_hbm, p2t_hbm, mask_hbm,
                    out_hbm,
                    r1n_v, r1d_v, mask_v, j1_v, m1_v, r2n_v, r2d_v, j2_v,
                    out_v):
    wid = lax.axis_index("s") * _SC_CORES + lax.axis_index("c")
    base = wid * _PER_W

    pltpu.sync_copy(d1t_hbm.at[0], r1n_v)
    pltpu.sync_copy(d1t_hbm.at[1], r1d_v)
    pltpu.sync_copy(mask_hbm, mask_v)
    pltpu.sync_copy(j1t_hbm.at[0], j1_v)
    pltpu.sync_copy(d2t_hbm.at[0, pl.ds(base, _PER_W)], r2n_v)
    pltpu.sync_copy(d2t_hbm.at[1, pl.ds(base, _PER_W)], r2d_v)
    pltpu.sync_copy(p2t_hbm.at[0, pl.ds(base, _PER_W)], j2_v)

    ratio_cut = jnp.float32(RATIO)
    zero = jnp.float32(0.0)
    one = jnp.float32(1.0)

    def m1_body(k, carry):
        sl = pl.ds(k * _SC_LANES, _SC_LANES)
        ratio = r1n_v[sl] / r1d_v[sl]
        res = jnp.where(ratio > ratio_cut, zero, ratio) * mask_v[sl]
        m1_v[sl] = jnp.where(res != zero, one, zero)
        return carry

    lax.fori_loop(0, L // _SC_LANES, m1_body, 0)

    lane_iota = lax.iota(jnp.int32, _SC_LANES)
    for s in range(_VREGS_PER_W):
        sl = pl.ds(s * _SC_LANES, _SC_LANES)
        j2s = j2_v[sl]
        g_m1 = plsc.load_gather(m1_v, [j2s])
        g_j1 = plsc.load_gather(j1_v, [j2s])
        ratio2 = r2n_v[sl] / r2d_v[sl]
        res2 = jnp.where(ratio2 > ratio_cut, zero, ratio2) * mask_v[
            pl.ds(base + s * _SC_LANES, _SC_LANES)]
        m2 = res2 != zero
        mut = m2 & (g_m1 != zero) & (g_j1 == (lane_iota + (base + s * _SC_LANES)))
        out_v[sl] = jnp.where(mut, 1, 0).astype(jnp.int32)

    pltpu.sync_copy(out_v, out_hbm.at[pl.ds(base, _PER_W)])


@functools.cache
def _mutual_sc():
    return pl.kernel(
        _mutual_sc_body,
        out_type=jax.ShapeDtypeStruct((L,), jnp.int32),
        mesh=plsc.VectorSubcoreMesh(core_axis_name="c", subcore_axis_name="s"),
        compiler_params=pltpu.CompilerParams(needs_layout_passes=False),
        scratch_types=[
            pltpu.VMEM((L,), jnp.float32),
            pltpu.VMEM((L,), jnp.float32),
            pltpu.VMEM((L,), jnp.float32),
            pltpu.VMEM((L,), jnp.int32),
            pltpu.VMEM((L,), jnp.float32),
            pltpu.VMEM((_PER_W,), jnp.float32),
            pltpu.VMEM((_PER_W,), jnp.float32),
            pltpu.VMEM((_PER_W,), jnp.int32),
            pltpu.VMEM((_PER_W,), jnp.int32),
        ],
    )


def kernel(feat_c0, feat_c1):
    scale = jnp.asarray(feat_c0.shape[-1], dtype=jnp.float32) ** 0.5
    f1r = feat_c0[0]
    f2r = feat_c1[0]
    f1 = f1r / scale
    f2 = f2r / scale
    n1 = jnp.sum(f1 * f1, axis=-1)
    n2 = jnp.sum(f2 * f2, axis=-1)

    distance1, preds1, d2t, p2t, d1t, j1t = _topk_call(f1r, f2r, n1, n2)
    distance2 = d2t.T
    preds2 = p2t.T

    mask = jnp.asarray(_border_mask_np())
    mutual_i32 = _mutual_sc()(d1t, j1t, d2t, p2t, mask)
    mutual = mutual_i32.astype(bool)
    return distance1, preds1, distance2, preds2, mutual


# speedup vs baseline: 59.7864x; 1.0661x over previous
import functools

import jax
import jax.numpy as jnp
import numpy as np
from jax import lax
from jax.experimental import pallas as pl
from jax.experimental.pallas import tpu as pltpu
from jax.experimental.pallas import tpu_sc as plsc

L = 4096
LENGTH = 64
C = 256
TOPK = 3
RATIO = 0.85

BR = 512
NB = L // BR
CH = L // 128
ST = BR // 8

_SC_CORES = 2
_SC_LANES = 16
_SC_WORKERS = 32
_PER_W = L // _SC_WORKERS
_VREGS_PER_W = _PER_W // _SC_LANES


def _border_mask_np():
    m = np.ones((LENGTH, LENGTH), dtype=np.float32)
    m[:2, :] = 0
    m[:, :2] = 0
    m[-2:, :] = 0
    m[:, -2:] = 0
    return m.reshape(-1)


def _insert(x, xi, v1, i1, v2, i2, v3, i3):
    c1 = x < v1
    c2 = x < v2
    c3 = x < v3
    v3n = jnp.where(c3, jnp.where(c2, v2, x), v3)
    i3n = jnp.where(c3, jnp.where(c2, i2, xi), i3)
    v2n = jnp.where(c2, jnp.where(c1, v1, x), v2)
    i2n = jnp.where(c2, jnp.where(c1, i1, xi), i2)
    v1n = jnp.where(c1, x, v1)
    i1n = jnp.where(c1, xi, i1)
    return v1n, i1n, v2n, i2n, v3n, i3n


def _extract3(vals, gidx, axis):
    INF = jnp.float32(jnp.inf)
    BIG = jnp.int32(2**30)
    out_v, out_i = [], []
    for k in range(TOPK):
        m = jnp.min(vals, axis=axis)
        me = jnp.expand_dims(m, axis)
        sel = jnp.min(jnp.where(vals == me, gidx, BIG), axis=axis)
        out_v.append(m)
        out_i.append(sel)
        if k < TOPK - 1:
            sele = jnp.expand_dims(sel, axis)
            vals = jnp.where(gidx == sele, INF, vals)
    return out_v, out_i


def _topk_body(f1_ref, f2_ref, n1_ref, n2_ref, maskb_ref, maskf_ref,
               d1_ref, p1_ref, d2t_ref, p2t_ref, j1t_ref, m1t_ref, m2t_ref,
               cval_ref, cidx_ref):
    i = pl.program_id(0)
    INF = jnp.float32(jnp.inf)
    base = i * BR

    f1 = f1_ref[...]
    n1col = n1_ref[...][:, None]
    n2 = n2_ref[...]
    lane128 = lax.broadcasted_iota(jnp.int32, (BR, 128), 1)
    sub8 = lax.broadcasted_iota(jnp.int32, (8, 128), 0) + base

    v1 = i1 = v2 = i2 = v3 = i3 = None
    col_v, col_i = [], []
    for c in range(CH):
        f2c = f2_ref[c * 128:(c + 1) * 128, :]
        g = lax.dot_general(f1, f2c, (((1,), (1,)), ((), ())),
                            preferred_element_type=jnp.float32)
        dc = (n1col + n2[None, c * 128:(c + 1) * 128]) - g * jnp.float32(2.0**-7)

        if c == 0:
            zero_i = jnp.zeros((BR, 128), jnp.int32)
            v1, i1 = dc, zero_i
            v2, i2 = jnp.full((BR, 128), INF), zero_i
            v3, i3 = jnp.full((BR, 128), INF), zero_i
        else:
            v1, i1, v2, i2, v3, i3 = _insert(dc, jnp.int32(c),
                                             v1, i1, v2, i2, v3, i3)

        zero_c = jnp.zeros((8, 128), jnp.int32)
        w1, k1 = dc[0:8, :], zero_c
        w2, k2 = jnp.full((8, 128), INF), zero_c
        w3, k3 = jnp.full((8, 128), INF), zero_c
        for s in range(1, ST):
            w1, k1, w2, k2, w3, k3 = _insert(dc[s * 8:(s + 1) * 8, :],
                                             jnp.int32(s),
                                             w1, k1, w2, k2, w3, k3)
        col_v.append((w1, w2, w3))
        col_i.append((k1 * 8 + sub8, k2 * 8 + sub8, k3 * 8 + sub8))

    rv = jnp.concatenate([v1, v2, v3], axis=1)
    rix = jnp.concatenate([i1 * 128 + lane128, i2 * 128 + lane128,
                           i3 * 128 + lane128], axis=1)
    vals, idxs = _extract3(rv, rix, axis=1)
    d1_ref[...] = jnp.stack(vals, axis=1)
    p1_ref[...] = jnp.stack(idxs, axis=1)
    j1t_ref[0, :] = idxs[0]
    ratio1 = vals[0] / vals[1]
    res1 = jnp.where(ratio1 > jnp.float32(RATIO), jnp.float32(0.0), ratio1)
    res1 = res1 * maskb_ref[...]
    m1t_ref[0, :] = jnp.where(res1 != 0.0, jnp.float32(1.0), jnp.float32(0.0))

    @pl.when(i == 0)
    def _():
        cval_ref[...] = jnp.full((TOPK, L), INF, jnp.float32)
        cidx_ref[...] = jnp.full((TOPK, L), jnp.int32(2**30), jnp.int32)

    wa = jnp.concatenate([w[0] for w in col_v], axis=1)
    wb = jnp.concatenate([w[1] for w in col_v], axis=1)
    wc = jnp.concatenate([w[2] for w in col_v], axis=1)
    ka = jnp.concatenate([k[0] for k in col_i], axis=1)
    kb = jnp.concatenate([k[1] for k in col_i], axis=1)
    kc = jnp.concatenate([k[2] for k in col_i], axis=1)
    cat_v = jnp.concatenate([cval_ref[...], wa, wb, wc], axis=0)
    cat_i = jnp.concatenate([cidx_ref[...], ka, kb, kc], axis=0)
    nv, ni = _extract3(cat_v, cat_i, axis=0)
    cval_ref[...] = jnp.stack(nv, axis=0)
    cidx_ref[...] = jnp.stack(ni, axis=0)

    @pl.when(i == NB - 1)
    def _():
        d2t_ref[...] = cval_ref[...]
        p2t_ref[...] = cidx_ref[...]
        ratio2 = cval_ref[0, :] / cval_ref[1, :]
        res2 = jnp.where(ratio2 > jnp.float32(RATIO), jnp.float32(0.0), ratio2)
        res2 = res2 * maskf_ref[...]
        m2t_ref[0, :] = jnp.where(res2 != 0.0, jnp.float32(1.0),
                                  jnp.float32(0.0))


_topk_call = pl.pallas_call(
    _topk_body,
    grid=(NB,),
    in_specs=[
        pl.BlockSpec((BR, C), lambda i: (i, 0)),
        pl.BlockSpec((L, C), lambda i: (0, 0)),
        pl.BlockSpec((BR,), lambda i: (i,)),
        pl.BlockSpec((L,), lambda i: (0,)),
        pl.BlockSpec((BR,), lambda i: (i,)),
        pl.BlockSpec((L,), lambda i: (0,)),
    ],
    out_specs=[
        pl.BlockSpec((BR, TOPK), lambda i: (i, 0)),
        pl.BlockSpec((BR, TOPK), lambda i: (i, 0)),
        pl.BlockSpec((TOPK, L), lambda i: (0, 0)),
        pl.BlockSpec((TOPK, L), lambda i: (0, 0)),
        pl.BlockSpec((1, BR), lambda i: (0, i)),
        pl.BlockSpec((1, BR), lambda i: (0, i)),
        pl.BlockSpec((1, L), lambda i: (0, 0)),
    ],
    out_shape=[
        jax.ShapeDtypeStruct((L, TOPK), jnp.float32),
        jax.ShapeDtypeStruct((L, TOPK), jnp.int32),
        jax.ShapeDtypeStruct((TOPK, L), jnp.float32),
        jax.ShapeDtypeStruct((TOPK, L), jnp.int32),
        jax.ShapeDtypeStruct((1, L), jnp.int32),
        jax.ShapeDtypeStruct((1, L), jnp.float32),
        jax.ShapeDtypeStruct((1, L), jnp.float32),
    ],
    scratch_shapes=[
        pltpu.VMEM((TOPK, L), jnp.float32),
        pltpu.VMEM((TOPK, L), jnp.int32),
    ],
    compiler_params=pltpu.CompilerParams(
        dimension_semantics=("arbitrary",),
    ),
)


def _mutual_sc_body(m1t_hbm, j1t_hbm, m2t_hbm, p2t_hbm,
                    out_hbm,
                    m1_v, j1_v, m2_v, j2_v, out_v):
    wid = lax.axis_index("s") * _SC_CORES + lax.axis_index("c")
    base = wid * _PER_W

    pltpu.sync_copy(m1t_hbm.at[0], m1_v)
    pltpu.sync_copy(j1t_hbm.at[0], j1_v)
    pltpu.sync_copy(m2t_hbm.at[0, pl.ds(base, _PER_W)], m2_v)
    pltpu.sync_copy(p2t_hbm.at[0, pl.ds(base, _PER_W)], j2_v)

    zero = jnp.float32(0.0)
    lane_iota = lax.iota(jnp.int32, _SC_LANES)
    for s in range(_VREGS_PER_W):
        sl = pl.ds(s * _SC_LANES, _SC_LANES)
        j2s = j2_v[sl]
        g_m1 = plsc.load_gather(m1_v, [j2s])
        g_j1 = plsc.load_gather(j1_v, [j2s])
        m2 = m2_v[sl] != zero
        mut = m2 & (g_m1 != zero) & (g_j1 == (lane_iota + (base + s * _SC_LANES)))
        out_v[sl] = jnp.where(mut, 1, 0).astype(jnp.int32)

    pltpu.sync_copy(out_v, out_hbm.at[pl.ds(base, _PER_W)])


@functools.cache
def _mutual_sc():
    return pl.kernel(
        _mutual_sc_body,
        out_type=jax.ShapeDtypeStruct((L,), jnp.int32),
        mesh=plsc.VectorSubcoreMesh(core_axis_name="c", subcore_axis_name="s"),
        compiler_params=pltpu.CompilerParams(needs_layout_passes=False),
        scratch_types=[
            pltpu.VMEM((L,), jnp.float32),
            pltpu.VMEM((L,), jnp.int32),
            pltpu.VMEM((_PER_W,), jnp.float32),
            pltpu.VMEM((_PER_W,), jnp.int32),
            pltpu.VMEM((_PER_W,), jnp.int32),
        ],
    )


def kernel(feat_c0, feat_c1):
    scale = jnp.asarray(feat_c0.shape[-1], dtype=jnp.float32) ** 0.5
    f1r = feat_c0[0]
    f2r = feat_c1[0]
    f1 = f1r / scale
    f2 = f2r / scale
    n1 = jnp.sum(f1 * f1, axis=-1)
    n2 = jnp.sum(f2 * f2, axis=-1)

    mask = jnp.asarray(_border_mask_np())
    distance1, preds1, d2t, p2t, j1t, m1t, m2t = _topk_call(
        f1r, f2r, n1, n2, mask, mask)
    distance2 = d2t.T
    preds2 = p2t.T

    mutual_i32 = _mutual_sc()(m1t, j1t, m2t, p2t)
    mutual = mutual_i32.astype(bool)
    return distance1, preds1, distance2, preds2, mutual
